# baseline (speedup 1.0000x reference)
import jax
import jax.numpy as jnp
from jax.experimental import pallas as pl
from jax.experimental.pallas import tpu as pltpu

NS = 32
NB_ANCH = 20000
NCLS = 81
A_BLK = 2000
NBLK = NB_ANCH // A_BLK


def _dense_body(pred_ref, gb_ref, tb_ref, tgt_ref,
                probs_ref, ce_ref, sl1_ref, cesum_ref, npos_ref):
    x = pred_ref[0]
    t = tgt_ref[0, 0]
    m = jnp.max(x, axis=1, keepdims=True)
    e = jnp.exp(x - m)
    se = jnp.sum(e, axis=1, keepdims=True)
    cls = jax.lax.broadcasted_iota(jnp.int32, x.shape, 1)
    xt = jnp.sum(jnp.where(cls == t, x, 0.0), axis=1, keepdims=True)
    ce = (m + jnp.log(se)) - xt
    bg = e[:, 0:1] / se
    pos = t > 0
    probs = jnp.where(pos, 1.0, bg)
    probs_ref[0, 0] = probs
    ce_ref[0, 0] = ce

    d = gb_ref[0] - tb_ref[0]
    ad = jnp.abs(d)
    sl1 = jnp.where(ad < 1.0, 0.5 * d * d, ad - 0.5)
    sla = jnp.sum(sl1, axis=1, keepdims=True)
    sl1_ref[0, 0] = jnp.sum(jnp.where(pos, sla, 0.0), axis=(0, 1),
                            keepdims=True)
    cesum_ref[0, 0] = jnp.sum(ce, axis=(0, 1), keepdims=True)
    npos_ref[0, 0] = jnp.sum(pos.astype(jnp.float32), axis=(0, 1),
                             keepdims=True)


def _mining_body(probs_ref, ce_ref, tgt_ref, sl1_ref, cesum_ref, nposp_ref,
                 out_ref):
    npos = jnp.sum(nposp_ref[...], axis=1, keepdims=True)
    retain_f = jnp.maximum(3.0 * npos, 1.0)
    all_fast = jnp.all(retain_f >= float(NB_ANCH))
    box_num = jnp.sum(sl1_ref[...], axis=(0, 1), keepdims=True)
    npos_tot = jnp.sum(npos, axis=(0, 1), keepdims=True)
    box_term = box_num / jnp.maximum(npos_tot, 1.0)

    @pl.when(all_fast)
    def _():
        det_num = jnp.sum(cesum_ref[...], axis=(0, 1), keepdims=True)
        det_cnt = float(NS * NB_ANCH)
        out_ref[...] = det_num / det_cnt + box_term

    @pl.when(jnp.logical_not(all_fast))
    def _():
        retain = jnp.minimum(retain_f, float(NB_ANCH)).astype(jnp.int32)
        bits = jax.lax.bitcast_convert_type(probs_ref[...], jnp.int32)
        col = jax.lax.broadcasted_iota(jnp.int32, bits.shape, 1)

        def bis_bits(_, lohi):
            lo, hi = lohi
            mid = lo + (hi - lo) // 2
            cnt = jnp.sum((bits <= mid).astype(jnp.int32), axis=1,
                          keepdims=True)
            p = cnt >= retain
            return jnp.where(p, lo, mid), jnp.where(p, mid, hi)

        lo0 = jnp.full((NS, 1), -1, jnp.int32)
        hi0 = jnp.full((NS, 1), 0x7F800000, jnp.int32)
        _, T = jax.lax.fori_loop(0, 31, bis_bits, (lo0, hi0))

        eq = bits == T
        cnt_less = jnp.sum((bits < T).astype(jnp.int32), axis=1,
                           keepdims=True)
        need_eq = retain - cnt_less

        def bis_idx(_, lohi):
            lo, hi = lohi
            mid = lo + (hi - lo) // 2
            cnt = jnp.sum((eq & (col < mid)).astype(jnp.int32), axis=1,
                          keepdims=True)
            p = cnt >= need_eq
            return jnp.where(p, lo, mid), jnp.where(p, mid, hi)

        lo0i = jnp.zeros((NS, 1), jnp.int32)
        hi0i = jnp.full((NS, 1), NB_ANCH, jnp.int32)
        _, I = jax.lax.fori_loop(0, 15, bis_idx, (lo0i, hi0i))

        pos = tgt_ref[...] > 0
        mask = pos | (bits < T) | (eq & (col < I))
        mf = mask.astype(jnp.float32)
        det_num = jnp.sum(ce_ref[...] * mf, axis=(0, 1), keepdims=True)
        det_cnt = jnp.sum(mf, axis=(0, 1), keepdims=True)
        out_ref[...] = det_num / jnp.maximum(det_cnt, 1.0) + box_term


def kernel(gcxcywh_boxes, predictions, target_boxes, targets):
    tgt4 = targets.reshape(NS, NBLK, A_BLK, 1)

    probs, ce, sl1p, cesump, nposp = pl.pallas_call(
        _dense_body,
        grid=(NS, NBLK),
        in_specs=[
            pl.BlockSpec((1, A_BLK, NCLS), lambda s, b: (s, b, 0)),
            pl.BlockSpec((1, A_BLK, 4), lambda s, b: (s, b, 0)),
            pl.BlockSpec((1, A_BLK, 4), lambda s, b: (s, b, 0)),
            pl.BlockSpec((1, 1, A_BLK, 1), lambda s, b: (s, b, 0, 0)),
        ],
        out_specs=[
            pl.BlockSpec((1, 1, A_BLK, 1), lambda s, b: (s, b, 0, 0)),
            pl.BlockSpec((1, 1, A_BLK, 1), lambda s, b: (s, b, 0, 0)),
            pl.BlockSpec((1, 1, 1, 1), lambda s, b: (s, b, 0, 0)),
            pl.BlockSpec((1, 1, 1, 1), lambda s, b: (s, b, 0, 0)),
            pl.BlockSpec((1, 1, 1, 1), lambda s, b: (s, b, 0, 0)),
        ],
        out_shape=[
            jax.ShapeDtypeStruct((NS, NBLK, A_BLK, 1), jnp.float32),
            jax.ShapeDtypeStruct((NS, NBLK, A_BLK, 1), jnp.float32),
            jax.ShapeDtypeStruct((NS, NBLK, 1, 1), jnp.float32),
            jax.ShapeDtypeStruct((NS, NBLK, 1, 1), jnp.float32),
            jax.ShapeDtypeStruct((NS, NBLK, 1, 1), jnp.float32),
        ],
        compiler_params=pltpu.CompilerParams(
            dimension_semantics=("parallel", "parallel")),
    )(predictions, gcxcywh_boxes, target_boxes, tgt4)

    probs2 = probs.reshape(NS, NB_ANCH)
    ce2 = ce.reshape(NS, NB_ANCH)
    sl1p2 = sl1p.reshape(NS, NBLK)
    cesump2 = cesump.reshape(NS, NBLK)
    nposp2 = nposp.reshape(NS, NBLK)

    out = pl.pallas_call(
        _mining_body,
        in_specs=[
            pl.BlockSpec((NS, NB_ANCH), lambda: (0, 0)),
            pl.BlockSpec((NS, NB_ANCH), lambda: (0, 0)),
            pl.BlockSpec((NS, NB_ANCH), lambda: (0, 0)),
            pl.BlockSpec((NS, NBLK), lambda: (0, 0)),
            pl.BlockSpec((NS, NBLK), lambda: (0, 0)),
            pl.BlockSpec((NS, NBLK), lambda: (0, 0)),
        ],
        out_specs=pl.BlockSpec((1, 1), lambda: (0, 0)),
        out_shape=jax.ShapeDtypeStruct((1, 1), jnp.float32),
    )(probs2, ce2, targets, sl1p2, cesump2, nposp2)

    return out[0, 0]


# speedup vs baseline: 2.0211x; 2.0211x over previous
import jax
import jax.numpy as jnp
from jax.experimental import pallas as pl
from jax.experimental.pallas import tpu as pltpu

NS = 32
NB_ANCH = 20000
NCLS = 81
A_BLK = 4000
NBLK = NB_ANCH // A_BLK


def _dense_body(pred_ref, gb_ref, tb_ref, tgt_ref,
                probs_ref, ce_ref, sl1_ref, cesum_ref, npos_ref):
    xT = pred_ref[0].T
    t = tgt_ref[0, 0]
    m = jnp.max(xT, axis=0, keepdims=True)
    e = jnp.exp(xT - m)
    se = jnp.sum(e, axis=0, keepdims=True)
    cls = jax.lax.broadcasted_iota(jnp.int32, xT.shape, 0)
    xt = jnp.sum(jnp.where(cls == t, xT, 0.0), axis=0, keepdims=True)
    ce = (m + jnp.log(se)) - xt
    bg = e[0:1, :] / se
    pos = t > 0
    posf = jnp.where(pos, 1.0, 0.0)
    probs = jnp.where(pos, 1.0, bg)
    probs_ref[0, 0] = probs
    ce_ref[0, 0] = ce

    d = gb_ref[0].T - tb_ref[0].T
    ad = jnp.abs(d)
    sl1 = jnp.where(ad < 1.0, 0.5 * d * d, ad - 0.5)
    sla = jnp.sum(sl1, axis=0, keepdims=True)
    sl1_ref[0, 0] = jnp.sum(sla * posf, axis=(0, 1), keepdims=True)
    cesum_ref[0, 0] = jnp.sum(ce, axis=(0, 1), keepdims=True)
    npos_ref[0, 0] = jnp.sum(posf, axis=(0, 1), keepdims=True)


def _mining_body(probs_ref, ce_ref, tgt_ref, sl1_ref, cesum_ref, nposp_ref,
                 out_ref):
    npos = jnp.sum(nposp_ref[...], axis=1, keepdims=True)
    retain_f = jnp.maximum(3.0 * npos, 1.0)
    all_fast = jnp.all(retain_f >= float(NB_ANCH))
    box_num = jnp.sum(sl1_ref[...], axis=(0, 1), keepdims=True)
    npos_tot = jnp.sum(npos, axis=(0, 1), keepdims=True)
    box_term = box_num / jnp.maximum(npos_tot, 1.0)

    @pl.when(all_fast)
    def _():
        det_num = jnp.sum(cesum_ref[...], axis=(0, 1), keepdims=True)
        det_cnt = float(NS * NB_ANCH)
        out_ref[...] = det_num / det_cnt + box_term

    @pl.when(jnp.logical_not(all_fast))
    def _():
        retain = jnp.minimum(retain_f, float(NB_ANCH)).astype(jnp.int32)
        bits = jax.lax.bitcast_convert_type(probs_ref[...], jnp.int32)
        col = jax.lax.broadcasted_iota(jnp.int32, bits.shape, 1)

        def bis_bits(_, lohi):
            lo, hi = lohi
            mid = lo + (hi - lo) // 2
            cnt = jnp.sum((bits <= mid).astype(jnp.int32), axis=1,
                          keepdims=True)
            p = cnt >= retain
            return jnp.where(p, lo, mid), jnp.where(p, mid, hi)

        lo0 = jnp.full((NS, 1), -1, jnp.int32)
        hi0 = jnp.full((NS, 1), 0x7F800000, jnp.int32)
        _, T = jax.lax.fori_loop(0, 31, bis_bits, (lo0, hi0))

        eq = bits == T
        cnt_less = jnp.sum((bits < T).astype(jnp.int32), axis=1,
                           keepdims=True)
        need_eq = retain - cnt_less

        def bis_idx(_, lohi):
            lo, hi = lohi
            mid = lo + (hi - lo) // 2
            cnt = jnp.sum((eq & (col < mid)).astype(jnp.int32), axis=1,
                          keepdims=True)
            p = cnt >= need_eq
            return jnp.where(p, lo, mid), jnp.where(p, mid, hi)

        lo0i = jnp.zeros((NS, 1), jnp.int32)
        hi0i = jnp.full((NS, 1), NB_ANCH, jnp.int32)
        _, I = jax.lax.fori_loop(0, 15, bis_idx, (lo0i, hi0i))

        pos = tgt_ref[...] > 0
        mask = pos | (bits < T) | (eq & (col < I))
        mf = mask.astype(jnp.float32)
        det_num = jnp.sum(ce_ref[...] * mf, axis=(0, 1), keepdims=True)
        det_cnt = jnp.sum(mf, axis=(0, 1), keepdims=True)
        out_ref[...] = det_num / jnp.maximum(det_cnt, 1.0) + box_term


def kernel(gcxcywh_boxes, predictions, target_boxes, targets):
    tgt4 = targets.reshape(NS, NBLK, 1, A_BLK)

    probs, ce, sl1p, cesump, nposp = pl.pallas_call(
        _dense_body,
        grid=(NS, NBLK),
        in_specs=[
            pl.BlockSpec((1, A_BLK, NCLS), lambda s, b: (s, b, 0)),
            pl.BlockSpec((1, A_BLK, 4), lambda s, b: (s, b, 0)),
            pl.BlockSpec((1, A_BLK, 4), lambda s, b: (s, b, 0)),
            pl.BlockSpec((1, 1, 1, A_BLK), lambda s, b: (s, b, 0, 0)),
        ],
        out_specs=[
            pl.BlockSpec((1, 1, 1, A_BLK), lambda s, b: (s, b, 0, 0)),
            pl.BlockSpec((1, 1, 1, A_BLK), lambda s, b: (s, b, 0, 0)),
            pl.BlockSpec((1, 1, 1, 1), lambda s, b: (s, b, 0, 0)),
            pl.BlockSpec((1, 1, 1, 1), lambda s, b: (s, b, 0, 0)),
            pl.BlockSpec((1, 1, 1, 1), lambda s, b: (s, b, 0, 0)),
        ],
        out_shape=[
            jax.ShapeDtypeStruct((NS, NBLK, 1, A_BLK), jnp.float32),
            jax.ShapeDtypeStruct((NS, NBLK, 1, A_BLK), jnp.float32),
            jax.ShapeDtypeStruct((NS, NBLK, 1, 1), jnp.float32),
            jax.ShapeDtypeStruct((NS, NBLK, 1, 1), jnp.float32),
            jax.ShapeDtypeStruct((NS, NBLK, 1, 1), jnp.float32),
        ],
        compiler_params=pltpu.CompilerParams(
            dimension_semantics=("parallel", "parallel")),
    )(predictions, gcxcywh_boxes, target_boxes, tgt4)

    probs2 = probs.reshape(NS, NB_ANCH)
    ce2 = ce.reshape(NS, NB_ANCH)
    sl1p2 = sl1p.reshape(NS, NBLK)
    cesump2 = cesump.reshape(NS, NBLK)
    nposp2 = nposp.reshape(NS, NBLK)

    out = pl.pallas_call(
        _mining_body,
        in_specs=[
            pl.BlockSpec((NS, NB_ANCH), lambda: (0, 0)),
            pl.BlockSpec((NS, NB_ANCH), lambda: (0, 0)),
            pl.BlockSpec((NS, NB_ANCH), lambda: (0, 0)),
            pl.BlockSpec((NS, NBLK), lambda: (0, 0)),
            pl.BlockSpec((NS, NBLK), lambda: (0, 0)),
            pl.BlockSpec((NS, NBLK), lambda: (0, 0)),
        ],
        out_specs=pl.BlockSpec((1, 1), lambda: (0, 0)),
        out_shape=jax.ShapeDtypeStruct((1, 1), jnp.float32),
    )(probs2, ce2, targets, sl1p2, cesump2, nposp2)

    return out[0, 0]
